# Initial kernel scaffold; baseline (speedup 1.0000x reference)
import jax
import jax.numpy as jnp
from jax.experimental import pallas as pl


def kernel(xyz, condi, featd, feat, params):
    raise NotImplementedError("write your pallas kernel here")


# speedup vs baseline: 1.1539x; 1.1539x over previous
import jax
import jax.numpy as jnp
from jax import lax
from jax.experimental import pallas as pl

_INTERPRET = False

B, N, CD = 2, 4096, 128
K, K8 = 16, 8
TQ = 512
T = N // TQ
GROUPS = 8
GC = CD // GROUPS
NCR = 16


def _leaky(x):
    return jnp.where(x >= 0, x, 0.1 * x)


def _sum_sq_rows(x):
    return (jnp.sum(x, axis=0, keepdims=True),
            jnp.sum(x * x, axis=0, keepdims=True))


def _acc_stats(ref, t, *pairs):
    @pl.when(t == 0)
    def _():
        ref[...] = jnp.zeros_like(ref)
    for i, (s, q) in enumerate(pairs):
        ref[0, 2 * i:2 * i + 1, :] += s
        ref[0, 2 * i + 1:2 * i + 2, :] += q


def _gn_consts(stats, cnt, gamma, beta):
    sg = jnp.sum(stats[0].reshape(GROUPS, GC), axis=1)
    qg = jnp.sum(stats[1].reshape(GROUPS, GC), axis=1)
    mu = sg / cnt
    var = qg / cnt - mu * mu
    s = gamma / jnp.sqrt(jnp.repeat(var, GC) + 1e-5)
    t = beta - jnp.repeat(mu, GC) * s
    return s, t


def _consts(*rows):
    rows = [jnp.broadcast_to(r, (B, CD)) for r in rows]
    rows += [jnp.zeros((B, CD), jnp.float32)] * (NCR - len(rows))
    return jnp.stack(rows, axis=1)


_W = pl.BlockSpec((CD, CD), lambda b, t: (0, 0))
_W16 = pl.BlockSpec((16, CD), lambda b, t: (0, 0))
_W256 = pl.BlockSpec((256, CD), lambda b, t: (0, 0))
_CONST = pl.BlockSpec((1, NCR, CD), lambda b, t: (b, 0, 0))
_GK = pl.BlockSpec((1, TQ, K, CD), lambda b, t: (b, t, 0, 0))
_GK8 = pl.BlockSpec((1, TQ, K8, CD), lambda b, t: (b, t, 0, 0))
_GX = pl.BlockSpec((1, TQ, K, 16), lambda b, t: (b, t, 0, 0))
_PT = pl.BlockSpec((1, TQ, CD), lambda b, t: (b, t, 0))
_PT16 = pl.BlockSpec((1, TQ, 16), lambda b, t: (b, t, 0))
_PT256 = pl.BlockSpec((1, TQ, 256), lambda b, t: (b, t, 0))
_STAT = pl.BlockSpec((1, 8, CD), lambda b, t: (b, 0, 0))

_STATSH = jax.ShapeDtypeStruct((B, 8, CD), jnp.float32)
_PTSH = jax.ShapeDtypeStruct((B, N, CD), jnp.float32)


def _pcall(body, in_specs, out_specs, out_shape):
    return pl.pallas_call(
        body, grid=(B, T), in_specs=in_specs, out_specs=out_specs,
        out_shape=out_shape, interpret=_INTERPRET)


def _rel_of(gx, xyztile):
    kk = gx.shape[1]
    rel = gx - xyztile[:, None, :]
    return rel.reshape(TQ * kk, 16)


def _dotf(a, b):
    return jnp.dot(a, b, preferred_element_type=jnp.float32)


def _cr(c_ref, i):
    return c_ref[0, i:i + 1, :]


def _p0_body(fa_ref, w1a_ref, f1_ref):
    f1_ref[0] = _dotf(fa_ref[0], w1a_ref[...])


def _a1_body(g1_ref, gx_ref, xyz_ref, w1b_ref, wp1_ref, c_ref,
             st1_ref, stp_ref):
    t = pl.program_id(1)
    rel = _rel_of(gx_ref[0], xyz_ref[0])
    h1 = (g1_ref[0].reshape(TQ * K, CD)
          + _dotf(rel, w1b_ref[...]) + _cr(c_ref, 0))
    p1 = _dotf(rel, wp1_ref[...]) + _cr(c_ref, 1)
    _acc_stats(st1_ref, t, _sum_sq_rows(h1))
    _acc_stats(stp_ref, t, _sum_sq_rows(p1))


def _a2_body(g1_ref, gx_ref, xyz_ref, w1b_ref, w2_ref, c_ref, st2_ref):
    t = pl.program_id(1)
    rel = _rel_of(gx_ref[0], xyz_ref[0])
    h1 = (g1_ref[0].reshape(TQ * K, CD)
          + _dotf(rel, w1b_ref[...]) + _cr(c_ref, 0))
    a = _leaky(h1 * _cr(c_ref, 1) + _cr(c_ref, 2))
    h2 = _dotf(a, w2_ref[...]) + _cr(c_ref, 3)
    _acc_stats(st2_ref, t, _sum_sq_rows(h2))


def _a3_body(g1_ref, gx_ref, xyz_ref, w1b_ref, w2_ref,
             wpre_ref, wq_ref, wk_ref, wv_ref, wa1_ref, c_ref,
             nf_ref, qa_ref, gk1_ref, fv_ref):
    rel = _rel_of(gx_ref[0], xyz_ref[0])
    h1 = (g1_ref[0].reshape(TQ * K, CD)
          + _dotf(rel, w1b_ref[...]) + _cr(c_ref, 0))
    a = _leaky(h1 * _cr(c_ref, 1) + _cr(c_ref, 2))
    h2 = _dotf(a, w2_ref[...]) + _cr(c_ref, 3)
    a2 = _leaky(h2 * _cr(c_ref, 4) + _cr(c_ref, 5))
    down = jnp.max(a2.reshape(TQ, K, CD), axis=1)
    nf = _dotf(down, wpre_ref[...]) + _cr(c_ref, 6)
    q = _dotf(nf, wq_ref[...]) + _cr(c_ref, 7)
    fk = _dotf(nf, wk_ref[...]) + _cr(c_ref, 8)
    fv = _dotf(nf, wv_ref[...]) + _cr(c_ref, 9)
    nf_ref[0] = nf
    qa_ref[0] = _dotf(q, wa1_ref[...])
    gk1_ref[0] = _dotf(fk, wa1_ref[...])
    fv_ref[0] = fv


def _att_pre(gk1_ref, gx_ref, xyz_ref, qa_ref, wp1_ref, wp2a_ref, c_ref):
    rel = _rel_of(gx_ref[0], xyz_ref[0])
    p1 = _dotf(rel, wp1_ref[...]) + _cr(c_ref, 0)
    ap = _leaky(p1 * _cr(c_ref, 1) + _cr(c_ref, 2))
    p2a = _dotf(ap, wp2a_ref[...])
    qa3 = qa_ref[0][:, None, :]
    a1 = qa3 + (p2a - gk1_ref[0].reshape(TQ * K, CD)
                + _cr(c_ref, 3)).reshape(TQ, K, CD)
    return a1.reshape(TQ * K, CD)


def _b2_body(gk1_ref, gx_ref, xyz_ref, qa_ref, wp1_ref, wp2a_ref, c_ref,
             sta_ref):
    t = pl.program_id(1)
    a1 = _att_pre(gk1_ref, gx_ref, xyz_ref, qa_ref, wp1_ref, wp2a_ref, c_ref)
    _acc_stats(sta_ref, t, _sum_sq_rows(a1))


def _b3_body(gk1_ref, gv_ref, gx_ref, xyz_ref, qa_ref, nf_ref,
             wp1_ref, wp2a_ref, wa2_ref, wpost_ref, c_ref,
             po_ref, stpo_ref):
    t = pl.program_id(1)
    a1 = _att_pre(gk1_ref, gx_ref, xyz_ref, qa_ref, wp1_ref, wp2a_ref, c_ref)
    ab = _leaky(a1 * _cr(c_ref, 4) + _cr(c_ref, 5))
    a2 = (_dotf(ab, wa2_ref[...]) + _cr(c_ref, 6)).reshape(TQ, K, CD)
    m = jnp.max(a2, axis=1, keepdims=True)
    e = jnp.exp(a2 - m)
    sm = e / jnp.sum(e, axis=1, keepdims=True)
    out = jnp.sum(sm * gv_ref[0], axis=1) + nf_ref[0]
    po = _dotf(out, wpost_ref[...]) + _cr(c_ref, 7)
    po_ref[0] = po
    _acc_stats(stpo_ref, t, _sum_sq_rows(po))


def _c0_body(po_ref, wu1a_ref, c_ref, o1_ref):
    out2 = _leaky(po_ref[0] * _cr(c_ref, 0) + _cr(c_ref, 1))
    o1_ref[0] = _dotf(out2, wu1a_ref[...])


def _c1_body(g3_ref, gx_ref, xyz_ref, wu1b_ref, c_ref, stu_ref):
    t = pl.program_id(1)
    rel8 = _rel_of(gx_ref[0][:, :K8, :], xyz_ref[0])
    u1 = (g3_ref[0].reshape(TQ * K8, CD)
          + _dotf(rel8, wu1b_ref[...]) + _cr(c_ref, 0))
    _acc_stats(stu_ref, t, _sum_sq_rows(u1))


def _c2_body(g3_ref, gx_ref, xyz_ref, featr_ref, wu1b_ref, wu2a_ref,
             wu2b_ref, c_ref, h_ref, sth_ref):
    t = pl.program_id(1)
    rel8 = _rel_of(gx_ref[0][:, :K8, :], xyz_ref[0])
    u1 = (g3_ref[0].reshape(TQ * K8, CD)
          + _dotf(rel8, wu1b_ref[...]) + _cr(c_ref, 0))
    au = _leaky(u1 * _cr(c_ref, 1) + _cr(c_ref, 2))
    um = jnp.max(au.reshape(TQ, K8, CD), axis=1)
    h = (_dotf(featr_ref[0], wu2a_ref[...])
         + _dotf(um, wu2b_ref[...]) + _cr(c_ref, 3))
    h_ref[0] = h
    _acc_stats(sth_ref, t, _sum_sq_rows(h))


def _c3_body(h_ref, c_ref, u_ref):
    u_ref[0] = _leaky(h_ref[0] * _cr(c_ref, 0) + _cr(c_ref, 1))


def _gather(tab, idx):
    return jax.vmap(lambda f, i: f[i])(tab, idx)


def kernel(xyz, condi, featd, feat, params):
    p = params
    xyzr = jnp.transpose(xyz, (0, 2, 1))
    xyz16 = jnp.pad(xyzr, ((0, 0), (0, 0), (0, 13)))
    featall_r = jnp.transpose(
        jnp.concatenate([condi, featd, feat], axis=1), (0, 2, 1))
    featr = jnp.transpose(feat, (0, 2, 1))

    ssq = jnp.sum(xyzr * xyzr, -1)
    d = (ssq[:, :, None] + ssq[:, None, :]
         - 2.0 * jnp.einsum('bqd,btd->bqt', xyzr, xyzr))
    _, idx = lax.top_k(-d, K)

    gxyz = _gather(xyz16, idx)

    w1b = jnp.pad(p['W1'][256:], ((0, 13), (0, 0)))
    wp1 = jnp.pad(p['Wp1'], ((0, 13), (0, 0)))
    wu1b = jnp.pad(p['Wu1'][128:], ((0, 13), (0, 0)))
    wp2a = jnp.dot(p['Wp2'], p['Wa1'])
    catt = jnp.dot(p['bp2'], p['Wa1']) + p['ba1']
    cnt_k = float(GC * N * K)
    cnt_k8 = float(GC * N * K8)
    cnt_n = float(GC * N)

    f1 = _pcall(_p0_body, [_PT256, _W256], _PT, _PTSH)(
        featall_r, p['W1'][:256])
    g1 = _gather(f1, idx)

    st1, stp = _pcall(
        _a1_body, [_GK, _GX, _PT16, _W16, _W16, _CONST],
        [_STAT, _STAT], [_STATSH, _STATSH])(
        g1, gxyz, xyz16, w1b, wp1, _consts(p['b1'], p['bp1']))
    s1, t1 = jax.vmap(lambda s: _gn_consts(s, cnt_k, p['g1'], p['be1']))(st1)
    sp, tp = jax.vmap(lambda s: _gn_consts(s, cnt_k, p['gp1'], p['bep1']))(stp)

    st2 = _pcall(
        _a2_body, [_GK, _GX, _PT16, _W16, _W, _CONST], _STAT, _STATSH)(
        g1, gxyz, xyz16, w1b, p['W2'], _consts(p['b1'], s1, t1, p['b2']))
    s2, t2 = jax.vmap(lambda s: _gn_consts(s, cnt_k, p['g2'], p['be2']))(st2)

    nf, qa, gk1, fv = _pcall(
        _a3_body,
        [_GK, _GX, _PT16, _W16, _W, _W, _W, _W, _W, _W, _CONST],
        [_PT, _PT, _PT, _PT], [_PTSH, _PTSH, _PTSH, _PTSH])(
        g1, gxyz, xyz16, w1b, p['W2'], p['Wpre'], p['Wq'], p['Wk'],
        p['Wv'], p['Wa1'],
        _consts(p['b1'], s1, t1, p['b2'], s2, t2, p['bpre'], p['bq'],
                p['bk'], p['bv']))

    ggk1 = _gather(gk1, idx)
    gv = _gather(fv, idx)

    sta = _pcall(
        _b2_body, [_GK, _GX, _PT16, _PT, _W16, _W, _CONST], _STAT, _STATSH)(
        ggk1, gxyz, xyz16, qa, wp1, wp2a, _consts(p['bp1'], sp, tp, catt))
    sa, ta = jax.vmap(lambda s: _gn_consts(s, cnt_k, p['ga1'], p['bea1']))(sta)

    po, stpo = _pcall(
        _b3_body,
        [_GK, _GK, _GX, _PT16, _PT, _PT, _W16, _W, _W, _W, _CONST],
        [_PT, _STAT], [_PTSH, _STATSH])(
        ggk1, gv, gxyz, xyz16, qa, nf, wp1, wp2a, p['Wa2'], p['Wpost'],
        _consts(p['bp1'], sp, tp, catt, sa, ta, p['ba2'], p['bpost']))
    spo, tpo = jax.vmap(
        lambda s: _gn_consts(s, cnt_n, p['gpost'], p['bepost']))(stpo)

    o1 = _pcall(_c0_body, [_PT, _W, _CONST], _PT, _PTSH)(
        po, p['Wu1'][:128], _consts(spo, tpo))
    g3 = _gather(o1, idx[:, :, :K8])

    stu = _pcall(
        _c1_body, [_GK8, _GX, _PT16, _W16, _CONST], _STAT, _STATSH)(
        g3, gxyz, xyz16, wu1b, _consts(p['bu1']))
    su, tu = jax.vmap(lambda s: _gn_consts(s, cnt_k8, p['gu1'], p['beu1']))(stu)

    h, sth = _pcall(
        _c2_body, [_GK8, _GX, _PT16, _PT, _W16, _W, _W, _CONST],
        [_PT, _STAT], [_PTSH, _STATSH])(
        g3, gxyz, xyz16, featr, wu1b, p['Wu2'][:128], p['Wu2'][128:],
        _consts(p['bu1'], su, tu, p['bu2']))
    sh, th = jax.vmap(lambda s: _gn_consts(s, cnt_n, p['gu2'], p['beu2']))(sth)

    u = _pcall(_c3_body, [_PT, _CONST], _PT, _PTSH)(h, _consts(sh, th))
    return jnp.transpose(u, (0, 2, 1))


# speedup vs baseline: 2.3301x; 2.0192x over previous
import functools

import jax
import jax.numpy as jnp
from jax import lax
from jax.experimental import pallas as pl
from jax.experimental.pallas import tpu as pltpu
from jax.experimental.pallas import tpu_sc as plsc

_INTERPRET = False

B, N, CD = 2, 4096, 128
K, K8 = 16, 8
TQ = 512
T = N // TQ
GROUPS = 8
GC = CD // GROUPS
NCR = 16


def _leaky(x):
    return jnp.where(x >= 0, x, 0.1 * x)


def _sum_sq_rows(x):
    return (jnp.sum(x, axis=0, keepdims=True),
            jnp.sum(x * x, axis=0, keepdims=True))


def _acc_stats(ref, t, *pairs):
    @pl.when(t == 0)
    def _():
        ref[...] = jnp.zeros_like(ref)
    for i, (s, q) in enumerate(pairs):
        ref[0, 2 * i:2 * i + 1, :] += s
        ref[0, 2 * i + 1:2 * i + 2, :] += q


def _gn_consts(stats, cnt, gamma, beta):
    sg = jnp.sum(stats[0].reshape(GROUPS, GC), axis=1)
    qg = jnp.sum(stats[1].reshape(GROUPS, GC), axis=1)
    mu = sg / cnt
    var = qg / cnt - mu * mu
    s = gamma / jnp.sqrt(jnp.repeat(var, GC) + 1e-5)
    t = beta - jnp.repeat(mu, GC) * s
    return s, t


def _consts(*rows):
    rows = [jnp.broadcast_to(r, (B, CD)) for r in rows]
    rows += [jnp.zeros((B, CD), jnp.float32)] * (NCR - len(rows))
    return jnp.stack(rows, axis=1)


_W = pl.BlockSpec((CD, CD), lambda b, t: (0, 0))
_W16 = pl.BlockSpec((16, CD), lambda b, t: (0, 0))
_W256 = pl.BlockSpec((256, CD), lambda b, t: (0, 0))
_CONST = pl.BlockSpec((1, NCR, CD), lambda b, t: (b, 0, 0))
_GK = pl.BlockSpec((1, TQ, K, CD), lambda b, t: (b, t, 0, 0))
_GK8 = pl.BlockSpec((1, TQ, K8, CD), lambda b, t: (b, t, 0, 0))
_GX = pl.BlockSpec((1, TQ, K, 16), lambda b, t: (b, t, 0, 0))
_PT = pl.BlockSpec((1, TQ, CD), lambda b, t: (b, t, 0))
_PT16 = pl.BlockSpec((1, TQ, 16), lambda b, t: (b, t, 0))
_PT256 = pl.BlockSpec((1, TQ, 256), lambda b, t: (b, t, 0))
_STAT = pl.BlockSpec((1, 8, CD), lambda b, t: (b, 0, 0))

_STATSH = jax.ShapeDtypeStruct((B, 8, CD), jnp.float32)
_PTSH = jax.ShapeDtypeStruct((B, N, CD), jnp.float32)


def _pcall(body, in_specs, out_specs, out_shape):
    return pl.pallas_call(
        body, grid=(B, T), in_specs=in_specs, out_specs=out_specs,
        out_shape=out_shape, interpret=_INTERPRET)


def _rel_of(gx, xyztile):
    kk = gx.shape[1]
    rel = gx - xyztile[:, None, :]
    return rel.reshape(TQ * kk, 16)


def _dotf(a, b):
    return jnp.dot(a, b, preferred_element_type=jnp.float32)


def _cr(c_ref, i):
    return c_ref[0, i:i + 1, :]


def _p0_body(fa_ref, w1a_ref, f1_ref):
    f1_ref[0] = _dotf(fa_ref[0], w1a_ref[...])


def _a1_body(g1_ref, gx_ref, xyz_ref, w1b_ref, wp1_ref, c_ref,
             st1_ref, stp_ref):
    t = pl.program_id(1)
    rel = _rel_of(gx_ref[0], xyz_ref[0])
    h1 = (g1_ref[0].reshape(TQ * K, CD)
          + _dotf(rel, w1b_ref[...]) + _cr(c_ref, 0))
    p1 = _dotf(rel, wp1_ref[...]) + _cr(c_ref, 1)
    _acc_stats(st1_ref, t, _sum_sq_rows(h1))
    _acc_stats(stp_ref, t, _sum_sq_rows(p1))


def _a2_body(g1_ref, gx_ref, xyz_ref, w1b_ref, w2_ref, c_ref, st2_ref):
    t = pl.program_id(1)
    rel = _rel_of(gx_ref[0], xyz_ref[0])
    h1 = (g1_ref[0].reshape(TQ * K, CD)
          + _dotf(rel, w1b_ref[...]) + _cr(c_ref, 0))
    a = _leaky(h1 * _cr(c_ref, 1) + _cr(c_ref, 2))
    h2 = _dotf(a, w2_ref[...]) + _cr(c_ref, 3)
    _acc_stats(st2_ref, t, _sum_sq_rows(h2))


def _a3_body(g1_ref, gx_ref, xyz_ref, w1b_ref, w2_ref,
             wpre_ref, wq_ref, wk_ref, wv_ref, wa1_ref, c_ref,
             nf_ref, qa_ref, gk1_ref, fv_ref):
    rel = _rel_of(gx_ref[0], xyz_ref[0])
    h1 = (g1_ref[0].reshape(TQ * K, CD)
          + _dotf(rel, w1b_ref[...]) + _cr(c_ref, 0))
    a = _leaky(h1 * _cr(c_ref, 1) + _cr(c_ref, 2))
    h2 = _dotf(a, w2_ref[...]) + _cr(c_ref, 3)
    a2 = _leaky(h2 * _cr(c_ref, 4) + _cr(c_ref, 5))
    down = jnp.max(a2.reshape(TQ, K, CD), axis=1)
    nf = _dotf(down, wpre_ref[...]) + _cr(c_ref, 6)
    q = _dotf(nf, wq_ref[...]) + _cr(c_ref, 7)
    fk = _dotf(nf, wk_ref[...]) + _cr(c_ref, 8)
    fv = _dotf(nf, wv_ref[...]) + _cr(c_ref, 9)
    nf_ref[0] = nf
    qa_ref[0] = _dotf(q, wa1_ref[...])
    gk1_ref[0] = _dotf(fk, wa1_ref[...])
    fv_ref[0] = fv


def _att_pre(gk1_ref, gx_ref, xyz_ref, qa_ref, wp1_ref, wp2a_ref, c_ref):
    rel = _rel_of(gx_ref[0], xyz_ref[0])
    p1 = _dotf(rel, wp1_ref[...]) + _cr(c_ref, 0)
    ap = _leaky(p1 * _cr(c_ref, 1) + _cr(c_ref, 2))
    p2a = _dotf(ap, wp2a_ref[...])
    qa3 = qa_ref[0][:, None, :]
    a1 = qa3 + (p2a - gk1_ref[0].reshape(TQ * K, CD)
                + _cr(c_ref, 3)).reshape(TQ, K, CD)
    return a1.reshape(TQ * K, CD)


def _b2_body(gk1_ref, gx_ref, xyz_ref, qa_ref, wp1_ref, wp2a_ref, c_ref,
             sta_ref):
    t = pl.program_id(1)
    a1 = _att_pre(gk1_ref, gx_ref, xyz_ref, qa_ref, wp1_ref, wp2a_ref, c_ref)
    _acc_stats(sta_ref, t, _sum_sq_rows(a1))


def _b3_body(gk1_ref, gv_ref, gx_ref, xyz_ref, qa_ref, nf_ref,
             wp1_ref, wp2a_ref, wa2_ref, wpost_ref, c_ref,
             po_ref, stpo_ref):
    t = pl.program_id(1)
    a1 = _att_pre(gk1_ref, gx_ref, xyz_ref, qa_ref, wp1_ref, wp2a_ref, c_ref)
    ab = _leaky(a1 * _cr(c_ref, 4) + _cr(c_ref, 5))
    a2 = (_dotf(ab, wa2_ref[...]) + _cr(c_ref, 6)).reshape(TQ, K, CD)
    m = jnp.max(a2, axis=1, keepdims=True)
    e = jnp.exp(a2 - m)
    sm = e / jnp.sum(e, axis=1, keepdims=True)
    out = jnp.sum(sm * gv_ref[0], axis=1) + nf_ref[0]
    po = _dotf(out, wpost_ref[...]) + _cr(c_ref, 7)
    po_ref[0] = po
    _acc_stats(stpo_ref, t, _sum_sq_rows(po))


def _c0_body(po_ref, wu1a_ref, c_ref, o1_ref):
    out2 = _leaky(po_ref[0] * _cr(c_ref, 0) + _cr(c_ref, 1))
    o1_ref[0] = _dotf(out2, wu1a_ref[...])


def _c1_body(g3_ref, gx_ref, xyz_ref, wu1b_ref, c_ref, stu_ref):
    t = pl.program_id(1)
    rel8 = _rel_of(gx_ref[0][:, :K8, :], xyz_ref[0])
    u1 = (g3_ref[0].reshape(TQ * K8, CD)
          + _dotf(rel8, wu1b_ref[...]) + _cr(c_ref, 0))
    _acc_stats(stu_ref, t, _sum_sq_rows(u1))


def _c2_body(g3_ref, gx_ref, xyz_ref, featr_ref, wu1b_ref, wu2a_ref,
             wu2b_ref, c_ref, h_ref, sth_ref):
    t = pl.program_id(1)
    rel8 = _rel_of(gx_ref[0][:, :K8, :], xyz_ref[0])
    u1 = (g3_ref[0].reshape(TQ * K8, CD)
          + _dotf(rel8, wu1b_ref[...]) + _cr(c_ref, 0))
    au = _leaky(u1 * _cr(c_ref, 1) + _cr(c_ref, 2))
    um = jnp.max(au.reshape(TQ, K8, CD), axis=1)
    h = (_dotf(featr_ref[0], wu2a_ref[...])
         + _dotf(um, wu2b_ref[...]) + _cr(c_ref, 3))
    h_ref[0] = h
    _acc_stats(sth_ref, t, _sum_sq_rows(h))


def _c3_body(h_ref, c_ref, u_ref):
    u_ref[0] = _leaky(h_ref[0] * _cr(c_ref, 0) + _cr(c_ref, 1))


def _sc_gather(table, idxg):
    V, D = table.shape
    M = idxg.shape[0]
    NW, CH = 32, 128
    mw = M // NW
    nch = mw // CH
    nbuf = 2
    idx3 = idxg.reshape(NW, nch, CH)
    mesh = plsc.VectorSubcoreMesh(core_axis_name="c", subcore_axis_name="s")

    @functools.partial(
        pl.kernel, mesh=mesh,
        compiler_params=pltpu.CompilerParams(use_tc_tiling_on_sc=(D % 128 == 0)),
        out_type=jax.ShapeDtypeStruct((M, D), jnp.float32),
        scratch_types=(
            [pltpu.VMEM((nch, CH), jnp.int32)]
            + [pltpu.VMEM((CH, D), jnp.float32) for _ in range(nbuf)]
            + [pltpu.SemaphoreType.DMA for _ in range(nbuf)]),
    )
    def k(table_hbm, idx_hbm, out_hbm, idx_v, *bufs_sems):
        bufs, sems = bufs_sems[:nbuf], bufs_sems[nbuf:]
        wid = lax.axis_index("s") * 2 + lax.axis_index("c")
        base = wid * mw
        pltpu.sync_copy(idx_hbm.at[wid], idx_v)
        for j in range(nbuf):
            pltpu.async_copy(table_hbm.at[idx_v.at[j]], bufs[j], sems[j])

        def outer(ci, _):
            for j in range(nbuf):
                c = ci * nbuf + j
                pltpu.make_async_copy(
                    table_hbm.at[idx_v.at[0]], bufs[j], sems[j]).wait()
                pltpu.sync_copy(bufs[j],
                                out_hbm.at[pl.ds(base + c * CH, CH)])
                nc = c + nbuf

                @pl.when(nc < nch)
                def _():
                    pltpu.async_copy(
                        table_hbm.at[idx_v.at[nc]], bufs[j], sems[j])
            return 0

        lax.fori_loop(0, nch // nbuf, outer, 0)

    return k(table, idx3)


def _gather(tab, idx):
    Bb, Nn, C = tab.shape
    k = idx.shape[-1]
    idxg = (idx + (jnp.arange(Bb, dtype=jnp.int32) * Nn)[:, None, None]
            ).reshape(Bb * Nn * k)
    out = _sc_gather(tab.reshape(Bb * Nn, C), idxg)
    return out.reshape(Bb, Nn, k, C)


def kernel(xyz, condi, featd, feat, params):
    p = params
    xyzr = jnp.transpose(xyz, (0, 2, 1))
    xyz16 = jnp.pad(xyzr, ((0, 0), (0, 0), (0, 13)))
    featall_r = jnp.transpose(
        jnp.concatenate([condi, featd, feat], axis=1), (0, 2, 1))
    featr = jnp.transpose(feat, (0, 2, 1))

    ssq = jnp.sum(xyzr * xyzr, -1)
    d = (ssq[:, :, None] + ssq[:, None, :]
         - 2.0 * jnp.einsum('bqd,btd->bqt', xyzr, xyzr))
    _, idx = lax.top_k(-d, K)

    gxyz = _gather(xyz16, idx)

    w1b = jnp.pad(p['W1'][256:], ((0, 13), (0, 0)))
    wp1 = jnp.pad(p['Wp1'], ((0, 13), (0, 0)))
    wu1b = jnp.pad(p['Wu1'][128:], ((0, 13), (0, 0)))
    wp2a = jnp.dot(p['Wp2'], p['Wa1'])
    catt = jnp.dot(p['bp2'], p['Wa1']) + p['ba1']
    cnt_k = float(GC * N * K)
    cnt_k8 = float(GC * N * K8)
    cnt_n = float(GC * N)

    f1 = _pcall(_p0_body, [_PT256, _W256], _PT, _PTSH)(
        featall_r, p['W1'][:256])
    g1 = _gather(f1, idx)

    st1, stp = _pcall(
        _a1_body, [_GK, _GX, _PT16, _W16, _W16, _CONST],
        [_STAT, _STAT], [_STATSH, _STATSH])(
        g1, gxyz, xyz16, w1b, wp1, _consts(p['b1'], p['bp1']))
    s1, t1 = jax.vmap(lambda s: _gn_consts(s, cnt_k, p['g1'], p['be1']))(st1)
    sp, tp = jax.vmap(lambda s: _gn_consts(s, cnt_k, p['gp1'], p['bep1']))(stp)

    st2 = _pcall(
        _a2_body, [_GK, _GX, _PT16, _W16, _W, _CONST], _STAT, _STATSH)(
        g1, gxyz, xyz16, w1b, p['W2'], _consts(p['b1'], s1, t1, p['b2']))
    s2, t2 = jax.vmap(lambda s: _gn_consts(s, cnt_k, p['g2'], p['be2']))(st2)

    nf, qa, gk1, fv = _pcall(
        _a3_body,
        [_GK, _GX, _PT16, _W16, _W, _W, _W, _W, _W, _W, _CONST],
        [_PT, _PT, _PT, _PT], [_PTSH, _PTSH, _PTSH, _PTSH])(
        g1, gxyz, xyz16, w1b, p['W2'], p['Wpre'], p['Wq'], p['Wk'],
        p['Wv'], p['Wa1'],
        _consts(p['b1'], s1, t1, p['b2'], s2, t2, p['bpre'], p['bq'],
                p['bk'], p['bv']))

    ggk1 = _gather(gk1, idx)
    gv = _gather(fv, idx)

    sta = _pcall(
        _b2_body, [_GK, _GX, _PT16, _PT, _W16, _W, _CONST], _STAT, _STATSH)(
        ggk1, gxyz, xyz16, qa, wp1, wp2a, _consts(p['bp1'], sp, tp, catt))
    sa, ta = jax.vmap(lambda s: _gn_consts(s, cnt_k, p['ga1'], p['bea1']))(sta)

    po, stpo = _pcall(
        _b3_body,
        [_GK, _GK, _GX, _PT16, _PT, _PT, _W16, _W, _W, _W, _CONST],
        [_PT, _STAT], [_PTSH, _STATSH])(
        ggk1, gv, gxyz, xyz16, qa, nf, wp1, wp2a, p['Wa2'], p['Wpost'],
        _consts(p['bp1'], sp, tp, catt, sa, ta, p['ba2'], p['bpost']))
    spo, tpo = jax.vmap(
        lambda s: _gn_consts(s, cnt_n, p['gpost'], p['bepost']))(stpo)

    o1 = _pcall(_c0_body, [_PT, _W, _CONST], _PT, _PTSH)(
        po, p['Wu1'][:128], _consts(spo, tpo))
    g3 = _gather(o1, idx[:, :, :K8])

    stu = _pcall(
        _c1_body, [_GK8, _GX, _PT16, _W16, _CONST], _STAT, _STATSH)(
        g3, gxyz, xyz16, wu1b, _consts(p['bu1']))
    su, tu = jax.vmap(lambda s: _gn_consts(s, cnt_k8, p['gu1'], p['beu1']))(stu)

    h, sth = _pcall(
        _c2_body, [_GK8, _GX, _PT16, _PT, _W16, _W, _W, _CONST],
        [_PT, _STAT], [_PTSH, _STATSH])(
        g3, gxyz, xyz16, featr, wu1b, p['Wu2'][:128], p['Wu2'][128:],
        _consts(p['bu1'], su, tu, p['bu2']))
    sh, th = jax.vmap(lambda s: _gn_consts(s, cnt_n, p['gu2'], p['beu2']))(sth)

    u = _pcall(_c3_body, [_PT, _CONST], _PT, _PTSH)(h, _consts(sh, th))
    return jnp.transpose(u, (0, 2, 1))


# speedup vs baseline: 12.9871x; 5.5737x over previous
import functools

import jax
import jax.numpy as jnp
from jax import lax
from jax.experimental import pallas as pl
from jax.experimental.pallas import tpu as pltpu
from jax.experimental.pallas import tpu_sc as plsc

_INTERPRET = False

B, N, CD = 2, 4096, 128
K, K8 = 16, 8
TQ = 512
T = N // TQ
GROUPS = 8
GC = CD // GROUPS
NCR = 16


def _leaky(x):
    return jnp.where(x >= 0, x, 0.1 * x)


def _sum_sq_rows(x):
    return (jnp.sum(x, axis=0, keepdims=True),
            jnp.sum(x * x, axis=0, keepdims=True))


def _acc_stats(ref, t, *pairs):
    @pl.when(t == 0)
    def _():
        ref[...] = jnp.zeros_like(ref)
    for i, (s, q) in enumerate(pairs):
        ref[0, 2 * i:2 * i + 1, :] += s
        ref[0, 2 * i + 1:2 * i + 2, :] += q


def _gn_consts(stats, cnt, gamma, beta):
    sg = jnp.sum(stats[0].reshape(GROUPS, GC), axis=1)
    qg = jnp.sum(stats[1].reshape(GROUPS, GC), axis=1)
    mu = sg / cnt
    var = qg / cnt - mu * mu
    s = gamma / jnp.sqrt(jnp.repeat(var, GC) + 1e-5)
    t = beta - jnp.repeat(mu, GC) * s
    return s, t


def _consts(*rows):
    rows = [jnp.broadcast_to(r, (B, CD)) for r in rows]
    rows += [jnp.zeros((B, CD), jnp.float32)] * (NCR - len(rows))
    return jnp.stack(rows, axis=1)


_W = pl.BlockSpec((CD, CD), lambda b, t: (0, 0))
_W16 = pl.BlockSpec((16, CD), lambda b, t: (0, 0))
_W256 = pl.BlockSpec((256, CD), lambda b, t: (0, 0))
_CONST = pl.BlockSpec((1, NCR, CD), lambda b, t: (b, 0, 0))
_GK = pl.BlockSpec((1, TQ, K, CD), lambda b, t: (b, t, 0, 0))
_GK8 = pl.BlockSpec((1, TQ, K8, CD), lambda b, t: (b, t, 0, 0))
_GX = pl.BlockSpec((1, TQ, K, 16), lambda b, t: (b, t, 0, 0))
_PT = pl.BlockSpec((1, TQ, CD), lambda b, t: (b, t, 0))
_PT16 = pl.BlockSpec((1, TQ, 16), lambda b, t: (b, t, 0))
_PT256 = pl.BlockSpec((1, TQ, 256), lambda b, t: (b, t, 0))
_STAT = pl.BlockSpec((1, 8, CD), lambda b, t: (b, 0, 0))

_STATSH = jax.ShapeDtypeStruct((B, 8, CD), jnp.float32)
_PTSH = jax.ShapeDtypeStruct((B, N, CD), jnp.float32)


def _pcall(body, in_specs, out_specs, out_shape):
    return pl.pallas_call(
        body, grid=(B, T), in_specs=in_specs, out_specs=out_specs,
        out_shape=out_shape, interpret=_INTERPRET)


def _rel_of(gx, xyztile):
    kk = gx.shape[1]
    rel = gx - xyztile[:, None, :]
    return rel.reshape(TQ * kk, 16)


def _dotf(a, b):
    return jnp.dot(a, b, preferred_element_type=jnp.float32)


def _cr(c_ref, i):
    return c_ref[0, i:i + 1, :]


def _knn_body(q_ref, tt_ref, idx_ref):
    Q = q_ref[0]
    Tt = tt_ref[0]
    qt = _dotf(Q, Tt)
    ssq_q = jnp.sum(Q * Q, axis=1, keepdims=True)
    ssq_t = jnp.sum(Tt * Tt, axis=0, keepdims=True)
    D = ssq_q + ssq_t - 2.0 * qt
    iota = lax.broadcasted_iota(jnp.int32, (TQ, N), 1)
    for k in range(K):
        m = jnp.min(D, axis=1, keepdims=True)
        ii = jnp.where(D == m, iota, N)
        a = jnp.min(ii, axis=1, keepdims=True)
        idx_ref[0, :, k:k + 1] = a
        D = jnp.where(ii == a, jnp.float32(3.4e38), D)


def _p0_body(fa_ref, w1a_ref, f1_ref):
    f1_ref[0] = _dotf(fa_ref[0], w1a_ref[...])


def _a1_body(g1_ref, gx_ref, xyz_ref, w1b_ref, wp1_ref, c_ref,
             st1_ref, stp_ref):
    t = pl.program_id(1)
    rel = _rel_of(gx_ref[0], xyz_ref[0])
    h1 = (g1_ref[0].reshape(TQ * K, CD)
          + _dotf(rel, w1b_ref[...]) + _cr(c_ref, 0))
    p1 = _dotf(rel, wp1_ref[...]) + _cr(c_ref, 1)
    _acc_stats(st1_ref, t, _sum_sq_rows(h1))
    _acc_stats(stp_ref, t, _sum_sq_rows(p1))


def _a2_body(g1_ref, gx_ref, xyz_ref, w1b_ref, w2_ref, c_ref, st2_ref):
    t = pl.program_id(1)
    rel = _rel_of(gx_ref[0], xyz_ref[0])
    h1 = (g1_ref[0].reshape(TQ * K, CD)
          + _dotf(rel, w1b_ref[...]) + _cr(c_ref, 0))
    a = _leaky(h1 * _cr(c_ref, 1) + _cr(c_ref, 2))
    h2 = _dotf(a, w2_ref[...]) + _cr(c_ref, 3)
    _acc_stats(st2_ref, t, _sum_sq_rows(h2))


def _a3_body(g1_ref, gx_ref, xyz_ref, w1b_ref, w2_ref,
             wpre_ref, wq_ref, wk_ref, wv_ref, wa1_ref, c_ref,
             nf_ref, qa_ref, gk1_ref, fv_ref):
    rel = _rel_of(gx_ref[0], xyz_ref[0])
    h1 = (g1_ref[0].reshape(TQ * K, CD)
          + _dotf(rel, w1b_ref[...]) + _cr(c_ref, 0))
    a = _leaky(h1 * _cr(c_ref, 1) + _cr(c_ref, 2))
    h2 = _dotf(a, w2_ref[...]) + _cr(c_ref, 3)
    a2 = _leaky(h2 * _cr(c_ref, 4) + _cr(c_ref, 5))
    down = jnp.max(a2.reshape(TQ, K, CD), axis=1)
    nf = _dotf(down, wpre_ref[...]) + _cr(c_ref, 6)
    q = _dotf(nf, wq_ref[...]) + _cr(c_ref, 7)
    fk = _dotf(nf, wk_ref[...]) + _cr(c_ref, 8)
    fv = _dotf(nf, wv_ref[...]) + _cr(c_ref, 9)
    nf_ref[0] = nf
    qa_ref[0] = _dotf(q, wa1_ref[...])
    gk1_ref[0] = _dotf(fk, wa1_ref[...])
    fv_ref[0] = fv


def _att_pre(gk1_ref, gx_ref, xyz_ref, qa_ref, wp1_ref, wp2a_ref, c_ref):
    rel = _rel_of(gx_ref[0], xyz_ref[0])
    p1 = _dotf(rel, wp1_ref[...]) + _cr(c_ref, 0)
    ap = _leaky(p1 * _cr(c_ref, 1) + _cr(c_ref, 2))
    p2a = _dotf(ap, wp2a_ref[...])
    qa3 = qa_ref[0][:, None, :]
    a1 = qa3 + (p2a - gk1_ref[0].reshape(TQ * K, CD)
                + _cr(c_ref, 3)).reshape(TQ, K, CD)
    return a1.reshape(TQ * K, CD)


def _b2_body(gk1_ref, gx_ref, xyz_ref, qa_ref, wp1_ref, wp2a_ref, c_ref,
             sta_ref):
    t = pl.program_id(1)
    a1 = _att_pre(gk1_ref, gx_ref, xyz_ref, qa_ref, wp1_ref, wp2a_ref, c_ref)
    _acc_stats(sta_ref, t, _sum_sq_rows(a1))


def _b3_body(gk1_ref, gv_ref, gx_ref, xyz_ref, qa_ref, nf_ref,
             wp1_ref, wp2a_ref, wa2_ref, wpost_ref, c_ref,
             po_ref, stpo_ref):
    t = pl.program_id(1)
    a1 = _att_pre(gk1_ref, gx_ref, xyz_ref, qa_ref, wp1_ref, wp2a_ref, c_ref)
    ab = _leaky(a1 * _cr(c_ref, 4) + _cr(c_ref, 5))
    a2 = (_dotf(ab, wa2_ref[...]) + _cr(c_ref, 6)).reshape(TQ, K, CD)
    m = jnp.max(a2, axis=1, keepdims=True)
    e = jnp.exp(a2 - m)
    sm = e / jnp.sum(e, axis=1, keepdims=True)
    out = jnp.sum(sm * gv_ref[0], axis=1) + nf_ref[0]
    po = _dotf(out, wpost_ref[...]) + _cr(c_ref, 7)
    po_ref[0] = po
    _acc_stats(stpo_ref, t, _sum_sq_rows(po))


def _c0_body(po_ref, wu1a_ref, c_ref, o1_ref):
    out2 = _leaky(po_ref[0] * _cr(c_ref, 0) + _cr(c_ref, 1))
    o1_ref[0] = _dotf(out2, wu1a_ref[...])


def _c1_body(g3_ref, gx_ref, xyz_ref, wu1b_ref, c_ref, stu_ref):
    t = pl.program_id(1)
    rel8 = _rel_of(gx_ref[0][:, :K8, :], xyz_ref[0])
    u1 = (g3_ref[0].reshape(TQ * K8, CD)
          + _dotf(rel8, wu1b_ref[...]) + _cr(c_ref, 0))
    _acc_stats(stu_ref, t, _sum_sq_rows(u1))


def _c2_body(g3_ref, gx_ref, xyz_ref, featr_ref, wu1b_ref, wu2a_ref,
             wu2b_ref, c_ref, h_ref, sth_ref):
    t = pl.program_id(1)
    rel8 = _rel_of(gx_ref[0][:, :K8, :], xyz_ref[0])
    u1 = (g3_ref[0].reshape(TQ * K8, CD)
          + _dotf(rel8, wu1b_ref[...]) + _cr(c_ref, 0))
    au = _leaky(u1 * _cr(c_ref, 1) + _cr(c_ref, 2))
    um = jnp.max(au.reshape(TQ, K8, CD), axis=1)
    h = (_dotf(featr_ref[0], wu2a_ref[...])
         + _dotf(um, wu2b_ref[...]) + _cr(c_ref, 3))
    h_ref[0] = h
    _acc_stats(sth_ref, t, _sum_sq_rows(h))


def _c3_body(h_ref, c_ref, u_ref):
    u_ref[0] = _leaky(h_ref[0] * _cr(c_ref, 0) + _cr(c_ref, 1))


def _sc_gather(table, idxg):
    V, D = table.shape
    M = idxg.shape[0]
    NW, CH = 32, 128
    mw = M // NW
    nch = mw // CH
    nbuf = 2
    idx3 = idxg.reshape(NW, nch, CH)
    mesh = plsc.VectorSubcoreMesh(core_axis_name="c", subcore_axis_name="s")

    @functools.partial(
        pl.kernel, mesh=mesh,
        compiler_params=pltpu.CompilerParams(use_tc_tiling_on_sc=(D % 128 == 0)),
        out_type=jax.ShapeDtypeStruct((M, D), jnp.float32),
        scratch_types=(
            [pltpu.VMEM((nch, CH), jnp.int32)]
            + [pltpu.VMEM((CH, D), jnp.float32) for _ in range(nbuf)]
            + [pltpu.SemaphoreType.DMA for _ in range(nbuf)]),
    )
    def k(table_hbm, idx_hbm, out_hbm, idx_v, *bufs_sems):
        bufs, sems = bufs_sems[:nbuf], bufs_sems[nbuf:]
        wid = lax.axis_index("s") * 2 + lax.axis_index("c")
        base = wid * mw
        pltpu.sync_copy(idx_hbm.at[wid], idx_v)
        for j in range(nbuf):
            pltpu.async_copy(table_hbm.at[idx_v.at[j]], bufs[j], sems[j])

        def outer(ci, _):
            for j in range(nbuf):
                c = ci * nbuf + j
                pltpu.make_async_copy(
                    table_hbm.at[idx_v.at[0]], bufs[j], sems[j]).wait()
                pltpu.sync_copy(bufs[j],
                                out_hbm.at[pl.ds(base + c * CH, CH)])
                nc = c + nbuf

                @pl.when(nc < nch)
                def _():
                    pltpu.async_copy(
                        table_hbm.at[idx_v.at[nc]], bufs[j], sems[j])
            return 0

        lax.fori_loop(0, nch // nbuf, outer, 0)

    return k(table, idx3)


def _gather(tab, idx):
    Bb, Nn, C = tab.shape
    k = idx.shape[-1]
    idxg = (idx + (jnp.arange(Bb, dtype=jnp.int32) * Nn)[:, None, None]
            ).reshape(Bb * Nn * k)
    out = _sc_gather(tab.reshape(Bb * Nn, C), idxg)
    return out.reshape(Bb, Nn, k, C)


def kernel(xyz, condi, featd, feat, params):
    p = params
    xyzr = jnp.transpose(xyz, (0, 2, 1))
    xyz16 = jnp.pad(xyzr, ((0, 0), (0, 0), (0, 13)))
    featall_r = jnp.transpose(
        jnp.concatenate([condi, featd, feat], axis=1), (0, 2, 1))
    featr = jnp.transpose(feat, (0, 2, 1))

    xyz16T = jnp.transpose(xyz16, (0, 2, 1))
    idx = pl.pallas_call(
        _knn_body, grid=(B, T),
        in_specs=[_PT16, pl.BlockSpec((1, 16, N), lambda b, t: (b, 0, 0))],
        out_specs=pl.BlockSpec((1, TQ, K), lambda b, t: (b, t, 0)),
        out_shape=jax.ShapeDtypeStruct((B, N, K), jnp.int32),
        interpret=_INTERPRET)(xyz16, xyz16T)

    gxyz = _gather(xyz16, idx)

    w1b = jnp.pad(p['W1'][256:], ((0, 13), (0, 0)))
    wp1 = jnp.pad(p['Wp1'], ((0, 13), (0, 0)))
    wu1b = jnp.pad(p['Wu1'][128:], ((0, 13), (0, 0)))
    wp2a = jnp.dot(p['Wp2'], p['Wa1'])
    catt = jnp.dot(p['bp2'], p['Wa1']) + p['ba1']
    cnt_k = float(GC * N * K)
    cnt_k8 = float(GC * N * K8)
    cnt_n = float(GC * N)

    f1 = _pcall(_p0_body, [_PT256, _W256], _PT, _PTSH)(
        featall_r, p['W1'][:256])
    g1 = _gather(f1, idx)

    st1, stp = _pcall(
        _a1_body, [_GK, _GX, _PT16, _W16, _W16, _CONST],
        [_STAT, _STAT], [_STATSH, _STATSH])(
        g1, gxyz, xyz16, w1b, wp1, _consts(p['b1'], p['bp1']))
    s1, t1 = jax.vmap(lambda s: _gn_consts(s, cnt_k, p['g1'], p['be1']))(st1)
    sp, tp = jax.vmap(lambda s: _gn_consts(s, cnt_k, p['gp1'], p['bep1']))(stp)

    st2 = _pcall(
        _a2_body, [_GK, _GX, _PT16, _W16, _W, _CONST], _STAT, _STATSH)(
        g1, gxyz, xyz16, w1b, p['W2'], _consts(p['b1'], s1, t1, p['b2']))
    s2, t2 = jax.vmap(lambda s: _gn_consts(s, cnt_k, p['g2'], p['be2']))(st2)

    nf, qa, gk1, fv = _pcall(
        _a3_body,
        [_GK, _GX, _PT16, _W16, _W, _W, _W, _W, _W, _W, _CONST],
        [_PT, _PT, _PT, _PT], [_PTSH, _PTSH, _PTSH, _PTSH])(
        g1, gxyz, xyz16, w1b, p['W2'], p['Wpre'], p['Wq'], p['Wk'],
        p['Wv'], p['Wa1'],
        _consts(p['b1'], s1, t1, p['b2'], s2, t2, p['bpre'], p['bq'],
                p['bk'], p['bv']))

    ggk1 = _gather(gk1, idx)
    gv = _gather(fv, idx)

    sta = _pcall(
        _b2_body, [_GK, _GX, _PT16, _PT, _W16, _W, _CONST], _STAT, _STATSH)(
        ggk1, gxyz, xyz16, qa, wp1, wp2a, _consts(p['bp1'], sp, tp, catt))
    sa, ta = jax.vmap(lambda s: _gn_consts(s, cnt_k, p['ga1'], p['bea1']))(sta)

    po, stpo = _pcall(
        _b3_body,
        [_GK, _GK, _GX, _PT16, _PT, _PT, _W16, _W, _W, _W, _CONST],
        [_PT, _STAT], [_PTSH, _STATSH])(
        ggk1, gv, gxyz, xyz16, qa, nf, wp1, wp2a, p['Wa2'], p['Wpost'],
        _consts(p['bp1'], sp, tp, catt, sa, ta, p['ba2'], p['bpost']))
    spo, tpo = jax.vmap(
        lambda s: _gn_consts(s, cnt_n, p['gpost'], p['bepost']))(stpo)

    o1 = _pcall(_c0_body, [_PT, _W, _CONST], _PT, _PTSH)(
        po, p['Wu1'][:128], _consts(spo, tpo))
    g3 = _gather(o1, idx[:, :, :K8])

    stu = _pcall(
        _c1_body, [_GK8, _GX, _PT16, _W16, _CONST], _STAT, _STATSH)(
        g3, gxyz, xyz16, wu1b, _consts(p['bu1']))
    su, tu = jax.vmap(lambda s: _gn_consts(s, cnt_k8, p['gu1'], p['beu1']))(stu)

    h, sth = _pcall(
        _c2_body, [_GK8, _GX, _PT16, _PT, _W16, _W, _W, _CONST],
        [_PT, _STAT], [_PTSH, _STATSH])(
        g3, gxyz, xyz16, featr, wu1b, p['Wu2'][:128], p['Wu2'][128:],
        _consts(p['bu1'], su, tu, p['bu2']))
    sh, th = jax.vmap(lambda s: _gn_consts(s, cnt_n, p['gu2'], p['beu2']))(sth)

    u = _pcall(_c3_body, [_PT, _CONST], _PT, _PTSH)(h, _consts(sh, th))
    return jnp.transpose(u, (0, 2, 1))
